# baseline (speedup 1.0000x reference)
import functools

import jax
import jax.numpy as jnp
from jax import lax
from jax.experimental import pallas as pl
from jax.experimental.pallas import tpu as pltpu
from jax.experimental.pallas import tpu_sc as plsc

NUM_EXPERTS = 8
D_MODEL = 1024
HIDDEN = 3 * D_MODEL
LANES = 128

ROUTER_CHUNK = 512
M = 256
D2 = D_MODEL // 2
NC, NS, L = 2, 16, 16
NW = NC * NS


def _rne_bf16_bits(f):
    bits = jax.lax.bitcast_convert_type(f, jnp.int32)
    rounded = bits + 0x7FFF + ((bits >> 16) & 1)
    return rounded & jnp.int32(-65536)


def _pack_halves(lo_f, hi_f):
    lo_bits = (_rne_bf16_bits(lo_f) >> 16) & jnp.int32(0xFFFF)
    return lo_bits | _rne_bf16_bits(hi_f)


def _unpack_lo(packed):
    return jax.lax.bitcast_convert_type(packed << 16, jnp.float32)


def _unpack_hi(packed):
    return jax.lax.bitcast_convert_type(
        packed & jnp.int32(-65536), jnp.float32)


def _router_body(x_ref, wr_ref, ltri_ref, mi_ref, mf_ref, cnt_ref, aux_ref,
                 xb_ref, carry_ref, psum_ref):
    xb_ref[...] = _pack_halves(x_ref[:, :D2], x_ref[:, D2:])
    i = pl.program_id(0)
    nsteps = pl.num_programs(0)
    ntok = x_ref.shape[0]

    logits = jnp.dot(x_ref[...], wr_ref[...], preferred_element_type=jnp.float32)
    col = jax.lax.broadcasted_iota(jnp.int32, logits.shape, 1)
    valid = col < NUM_EXPERTS
    logits = jnp.where(valid, logits, -jnp.inf)
    mx = jnp.max(logits, axis=1, keepdims=True)
    ex = jnp.where(valid, jnp.exp(logits - mx), 0.0)
    sm = jnp.sum(ex, axis=1, keepdims=True)
    probs = ex / sm

    v1 = jnp.max(probs, axis=1, keepdims=True)
    i1 = jnp.min(jnp.where((probs == v1) & valid, col, LANES), axis=1, keepdims=True)
    m1 = col == i1
    probs2 = jnp.where(m1, -1.0, probs)
    v2 = jnp.max(probs2, axis=1, keepdims=True)
    i2 = jnp.min(jnp.where((probs2 == v2) & valid, col, LANES), axis=1, keepdims=True)
    m2 = col == i2

    denom = v1 + v2
    w0 = v1 / denom
    w1 = v2 / denom

    @pl.when(i == 0)
    def _():
        carry_ref[...] = jnp.zeros((1, LANES), jnp.float32)
        psum_ref[...] = jnp.zeros((1, LANES), jnp.float32)

    oh0 = m1.astype(jnp.float32)
    oh1 = m2.astype(jnp.float32)
    ohsum = oh0 + oh1
    s_cum = jnp.dot(ltri_ref[...], ohsum, preferred_element_type=jnp.float32)

    carry = carry_ref[...]
    rank0 = jnp.sum((s_cum + carry) * oh0, axis=1, keepdims=True).astype(jnp.int32)
    rank1 = jnp.sum((s_cum + carry) * oh1, axis=1, keepdims=True).astype(jnp.int32)
    carry_ref[...] = carry + jnp.sum(ohsum, axis=0, keepdims=True)
    psum_ref[...] += jnp.sum(probs, axis=0, keepdims=True)

    er0 = (i1 << 13) | rank0
    er1 = (i2 << 13) | rank1
    mi_ref[...] = jnp.where(col == 0, er0, 0) + jnp.where(col == 1, er1, 0)
    mf_ref[...] = jnp.where(col == 0, w0, 0.0) + jnp.where(col == 1, w1, 0.0)

    @pl.when(i == nsteps - 1)
    def _():
        total = ntok * nsteps
        cnt = carry_ref[...]
        cnt_ref[...] = cnt
        fi = cnt / total
        pi = psum_ref[...] / total
        aux = 0.01 * NUM_EXPERTS * jnp.sum(fi * pi)
        aux_ref[...] = jnp.full((1, LANES), aux, dtype=jnp.float32)


def _router(x_flat, wr_pad, ltri):
    n = x_flat.shape[0]
    nsteps = n // ROUTER_CHUNK
    return pl.pallas_call(
        _router_body,
        grid=(nsteps,),
        in_specs=[
            pl.BlockSpec((ROUTER_CHUNK, D_MODEL), lambda i: (i, 0)),
            pl.BlockSpec((D_MODEL, LANES), lambda i: (0, 0)),
            pl.BlockSpec((ROUTER_CHUNK, ROUTER_CHUNK), lambda i: (0, 0)),
        ],
        out_specs=[
            pl.BlockSpec((ROUTER_CHUNK, LANES), lambda i: (i, 0)),
            pl.BlockSpec((ROUTER_CHUNK, LANES), lambda i: (i, 0)),
            pl.BlockSpec((1, LANES), lambda i: (0, 0)),
            pl.BlockSpec((1, LANES), lambda i: (0, 0)),
            pl.BlockSpec((ROUTER_CHUNK, D2), lambda i: (i, 0)),
        ],
        out_shape=[
            jax.ShapeDtypeStruct((n, LANES), jnp.int32),
            jax.ShapeDtypeStruct((n, LANES), jnp.float32),
            jax.ShapeDtypeStruct((1, LANES), jnp.float32),
            jax.ShapeDtypeStruct((1, LANES), jnp.float32),
            jax.ShapeDtypeStruct((n, D2), jnp.int32),
        ],
        scratch_shapes=[
            pltpu.VMEM((1, LANES), jnp.float32),
            pltpu.VMEM((1, LANES), jnp.float32),
        ],
    )(x_flat, wr_pad, ltri)


def _dispatch(x_flat, er0, er1, off16):
    n = x_flat.shape[0]
    tpw = n // NW
    sub = 64

    @functools.partial(
        pl.kernel,
        out_type=[
            jax.ShapeDtypeStruct((2 * n, D2), jnp.int32),
            jax.ShapeDtypeStruct((n,), jnp.int32),
            jax.ShapeDtypeStruct((n,), jnp.int32),
        ],
        mesh=plsc.VectorSubcoreMesh(core_axis_name="c", subcore_axis_name="s"),
        scratch_types=[
            pltpu.VMEM((16,), jnp.int32),
            pltpu.VMEM((sub,), jnp.int32),
            pltpu.VMEM((sub,), jnp.int32),
            pltpu.VMEM((sub,), jnp.int32),
            pltpu.VMEM((sub,), jnp.int32),
            pltpu.VMEM((sub, D2), jnp.int32),
            pltpu.SemaphoreType.DMA,
            pltpu.SemaphoreType.DMA,
            pltpu.SemaphoreType.DMA,
        ],
    )
    def k(x_hbm, er0_hbm, er1_hbm, offs_hbm,
          xs_hbm, p0_hbm, p1_hbm,
          off_v, er0_v, er1_v, idx0_v, idx1_v, xv, sem0, sem1, semx):
        wid = lax.axis_index("s") * NC + lax.axis_index("c")
        base = wid * tpw
        pltpu.sync_copy(offs_hbm, off_v)
        for j in range(tpw // sub):
            tb = base + j * sub
            cpx = pltpu.async_copy(x_hbm.at[pl.ds(tb, sub)], xv, semx)
            cp0 = pltpu.async_copy(er0_hbm.at[pl.ds(tb, sub)], er0_v, sem0)
            cp1 = pltpu.async_copy(er1_hbm.at[pl.ds(tb, sub)], er1_v, sem1)
            cp0.wait()
            cp1.wait()
            off_reg = off_v[...]
            for s in range(sub // L):
                sl = pl.ds(s * L, L)
                erv0 = er0_v[sl]
                erv1 = er1_v[sl]
                ev0 = erv0 >> 13
                ev1 = erv1 >> 13
                acc0 = erv0 & 8191
                acc1 = erv1 & 8191
                for kk in range(NUM_EXPERTS):
                    off_k = off_reg[kk]
                    acc0 = acc0 + jnp.where(ev0 == kk, off_k, 0)
                    acc1 = acc1 + jnp.where(ev1 == kk, off_k, 0)
                idx0_v[sl] = acc0
                idx1_v[sl] = acc1
            cpx.wait()
            cs0 = pltpu.async_copy(xv, xs_hbm.at[idx0_v], sem0)
            cs1 = pltpu.async_copy(xv, xs_hbm.at[idx1_v], sem1)
            cs0.wait()
            cs1.wait()
            pltpu.sync_copy(idx0_v, p0_hbm.at[pl.ds(tb, sub)])
            pltpu.sync_copy(idx1_v, p1_hbm.at[pl.ds(tb, sub)])

    return k(x_flat, er0, er1, off16)


H2 = HIDDEN // 2


def _gffn_half(xs32, w1, w2, step_e, step_m, step_act, off9, nt, half, prev):
    rows = xs32.shape[0]

    def body(se_ref, sm_ref, sa_ref, off_ref, xs_ref, w1_ref, w2_ref, *rest):
        if prev is not None:
            yp_ref, out_ref, acc_ref = rest
        else:
            out_ref, acc_ref = rest
        g = pl.program_id(0)
        nsteps = pl.num_programs(0)
        e = se_ref[g]
        m = sm_ref[g]
        prev_m = sm_ref[jnp.maximum(g - 1, 0)]
        first = (g == 0) | (m != prev_m)
        gn = jnp.minimum(g + 1, nsteps - 1)
        last = (g == nsteps - 1) | (sm_ref[gn] != m) | (sa_ref[gn] == 0)

        @pl.when(sa_ref[g] == 1)
        def _():
            packed = xs_ref[...]
            a = _unpack_lo(packed)
            bzz = _unpack_hi(packed)
            h = (jnp.dot(a, w1_ref[0, :D2, :], preferred_element_type=jnp.float32)
                 + jnp.dot(bzz, w1_ref[0, D2:, :],
                           preferred_element_type=jnp.float32))
            h = h * jax.nn.sigmoid(h)
            o = jnp.dot(h, w2_ref[0], preferred_element_type=jnp.float32)
            rowg = m * M + jax.lax.broadcasted_iota(jnp.int32, (M, 1), 0)
            mask = (rowg >= off_ref[e]) & (rowg < off_ref[e + 1])
            contrib = jnp.where(mask, o, 0.0)

            @pl.when(first)
            def _():
                acc_ref[...] = contrib

            @pl.when(jnp.logical_not(first))
            def _():
                acc_ref[...] += contrib

            @pl.when(last)
            def _():
                acc = acc_ref[...]
                lo = acc[:, :D2]
                hi = acc[:, D2:]
                if prev is not None:
                    yp = yp_ref[...]
                    lo = lo + _unpack_lo(yp)
                    hi = hi + _unpack_hi(yp)
                out_ref[...] = _pack_halves(lo, hi)

    in_specs = [
        pl.BlockSpec((M, D2), lambda g, se, sm, sa, off: (sm[g], 0)),
        pl.BlockSpec((1, D_MODEL, H2),
                     lambda g, se, sm, sa, off: (se[g], 0, half)),
        pl.BlockSpec((1, H2, D_MODEL),
                     lambda g, se, sm, sa, off: (se[g], half, 0)),
    ]
    args = [step_e, step_m, step_act, off9, xs32, w1, w2]
    if prev is not None:
        in_specs.append(pl.BlockSpec((M, D2), lambda g, se, sm, sa, off: (sm[g], 0)))
        args.append(prev)
    grid_spec = pltpu.PrefetchScalarGridSpec(
        num_scalar_prefetch=4,
        grid=(nt,),
        in_specs=in_specs,
        out_specs=pl.BlockSpec((M, D2), lambda g, se, sm, sa, off: (sm[g], 0)),
        scratch_shapes=[pltpu.VMEM((M, D_MODEL), jnp.float32)],
    )
    return pl.pallas_call(
        body,
        grid_spec=grid_spec,
        out_shape=jax.ShapeDtypeStruct((rows, D2), jnp.int32),
    )(*args)


def _gffn(xs32, w1, w2, step_e, step_m, step_act, off9, nt):
    ys0 = _gffn_half(xs32, w1, w2, step_e, step_m, step_act, off9, nt, 0, None)
    return _gffn_half(xs32, w1, w2, step_e, step_m, step_act, off9, nt, 1, ys0)


def _permute(ys, p0, p1):
    n = p0.shape[0]
    tpw = n // NW
    sub = 64

    @functools.partial(
        pl.kernel,
        out_type=[
            jax.ShapeDtypeStruct((n, D2), jnp.int32),
            jax.ShapeDtypeStruct((n, D2), jnp.int32),
        ],
        mesh=plsc.VectorSubcoreMesh(core_axis_name="c", subcore_axis_name="s"),
        scratch_types=[
            pltpu.VMEM((sub,), jnp.int32),
            pltpu.VMEM((sub,), jnp.int32),
            pltpu.VMEM((sub, D2), jnp.int32),
            pltpu.VMEM((sub, D2), jnp.int32),
            pltpu.SemaphoreType.DMA,
            pltpu.SemaphoreType.DMA,
        ],
    )
    def k(ys_hbm, p0_hbm, p1_hbm, ya_hbm, yb_hbm,
          p0_v, p1_v, ya, yb, sem0, sem1):
        wid = lax.axis_index("s") * NC + lax.axis_index("c")
        base = wid * tpw
        for j in range(tpw // sub):
            tb = base + j * sub
            pltpu.sync_copy(p0_hbm.at[pl.ds(tb, sub)], p0_v)
            pltpu.sync_copy(p1_hbm.at[pl.ds(tb, sub)], p1_v)
            cpa = pltpu.async_copy(ys_hbm.at[p0_v], ya, sem0)
            cpb = pltpu.async_copy(ys_hbm.at[p1_v], yb, sem1)
            cpa.wait()
            cpb.wait()
            pltpu.sync_copy(ya, ya_hbm.at[pl.ds(tb, sub)])
            pltpu.sync_copy(yb, yb_hbm.at[pl.ds(tb, sub)])

    return k(ys, p0, p1)


def _wsum_body(ya_ref, yb_ref, mf_ref, out_ref):
    w0 = mf_ref[:, 0:1]
    w1 = mf_ref[:, 1:2]
    ya = ya_ref[...]
    yb = yb_ref[...]
    out_ref[:, :D2] = _unpack_lo(ya) * w0 + _unpack_lo(yb) * w1
    out_ref[:, D2:] = _unpack_hi(ya) * w0 + _unpack_hi(yb) * w1


def _wsum(ya32, yb32, mf):
    n = ya32.shape[0]
    chunk = 512
    return pl.pallas_call(
        _wsum_body,
        grid=(n // chunk,),
        in_specs=[
            pl.BlockSpec((chunk, D2), lambda i: (i, 0)),
            pl.BlockSpec((chunk, D2), lambda i: (i, 0)),
            pl.BlockSpec((chunk, LANES), lambda i: (i, 0)),
        ],
        out_specs=pl.BlockSpec((chunk, D_MODEL), lambda i: (i, 0)),
        out_shape=jax.ShapeDtypeStruct((n, D_MODEL), jnp.float32),
    )(ya32, yb32, mf)


def kernel(x, Wr, W1, W2):
    b, t, c = x.shape
    n = b * t
    x_flat = x.reshape(n, c)
    wr_pad = jnp.pad(Wr, ((0, 0), (0, LANES - NUM_EXPERTS)))

    ltri = jnp.tril(jnp.ones((ROUTER_CHUNK, ROUTER_CHUNK), jnp.float32), -1)
    mi, mf, cnt_row, aux_vec, xb = _router(x_flat, wr_pad, ltri)
    er0 = mi[:, 0]
    er1 = mi[:, 1]
    counts = cnt_row[0, :NUM_EXPERTS].astype(jnp.int32)

    off9 = jnp.concatenate([jnp.zeros((1,), jnp.int32), jnp.cumsum(counts)])
    off16 = jnp.pad(off9, (0, 16 - off9.shape[0]))

    mt = (2 * n) // M
    nt = mt + NUM_EXPERTS - 1
    lo_row = off9[:NUM_EXPERTS]
    hi_row = off9[1:]
    tile_lo = lo_row // M
    tile_last = jnp.where(counts > 0, (hi_row - 1) // M, tile_lo)
    nact = jnp.where(counts > 0, tile_last - tile_lo + 1, 0)
    gstart = jnp.concatenate([jnp.zeros((1,), jnp.int32),
                              jnp.cumsum(nact)[:-1]])
    g = jnp.arange(nt, dtype=jnp.int32)
    e_of_g = jnp.sum((g[:, None] >= gstart[None, :]).astype(jnp.int32), axis=1) - 1
    within = g - gstart[e_of_g]
    act = within < nact[e_of_g]
    m_of_g = jnp.where(act, tile_lo[e_of_g] + within, mt - 1)
    step_e = e_of_g.astype(jnp.int32)
    step_m = m_of_g.astype(jnp.int32)
    step_act = act.astype(jnp.int32)

    xs32, p0, p1 = _dispatch(xb, er0, er1, off16)
    ys32 = _gffn(xs32, W1, W2, step_e, step_m, step_act, off9, nt)
    ya32, yb32 = _permute(ys32, p0, p1)
    out_flat = _wsum(ya32, yb32, mf)
    return out_flat.reshape(b, t, c), aux_vec[0, 0]


# speedup vs baseline: 1.1423x; 1.1423x over previous
import functools

import jax
import jax.numpy as jnp
from jax import lax
from jax.experimental import pallas as pl
from jax.experimental.pallas import tpu as pltpu
from jax.experimental.pallas import tpu_sc as plsc

NUM_EXPERTS = 8
D_MODEL = 1024
HIDDEN = 3 * D_MODEL
LANES = 128

ROUTER_CHUNK = 512
M = 256
D2 = D_MODEL // 2
NC, NS, L = 2, 16, 16
NW = NC * NS


def _rne_bf16_bits(f):
    bits = jax.lax.bitcast_convert_type(f, jnp.int32)
    rounded = bits + 0x7FFF + ((bits >> 16) & 1)
    return rounded & jnp.int32(-65536)


def _pack_halves(lo_f, hi_f):
    lo_bits = (_rne_bf16_bits(lo_f) >> 16) & jnp.int32(0xFFFF)
    return lo_bits | _rne_bf16_bits(hi_f)


def _unpack_lo(packed):
    return jax.lax.bitcast_convert_type(packed << 16, jnp.float32)


def _unpack_hi(packed):
    return jax.lax.bitcast_convert_type(
        packed & jnp.int32(-65536), jnp.float32)


def _router_body(x_ref, wr_ref, ltri_ref, mi_ref, mf_ref, cnt_ref, aux_ref,
                 xb_ref, carry_ref, psum_ref):
    xb_ref[...] = _pack_halves(x_ref[:, :D2], x_ref[:, D2:])
    i = pl.program_id(0)
    nsteps = pl.num_programs(0)
    ntok = x_ref.shape[0]

    logits = jnp.dot(x_ref[...], wr_ref[...], preferred_element_type=jnp.float32)
    col = jax.lax.broadcasted_iota(jnp.int32, logits.shape, 1)
    valid = col < NUM_EXPERTS
    logits = jnp.where(valid, logits, -jnp.inf)
    mx = jnp.max(logits, axis=1, keepdims=True)
    ex = jnp.where(valid, jnp.exp(logits - mx), 0.0)
    sm = jnp.sum(ex, axis=1, keepdims=True)
    probs = ex / sm

    v1 = jnp.max(probs, axis=1, keepdims=True)
    i1 = jnp.min(jnp.where((probs == v1) & valid, col, LANES), axis=1, keepdims=True)
    m1 = col == i1
    probs2 = jnp.where(m1, -1.0, probs)
    v2 = jnp.max(probs2, axis=1, keepdims=True)
    i2 = jnp.min(jnp.where((probs2 == v2) & valid, col, LANES), axis=1, keepdims=True)
    m2 = col == i2

    denom = v1 + v2
    w0 = v1 / denom
    w1 = v2 / denom

    @pl.when(i == 0)
    def _():
        carry_ref[...] = jnp.zeros((1, LANES), jnp.float32)
        psum_ref[...] = jnp.zeros((1, LANES), jnp.float32)

    oh0 = m1.astype(jnp.float32)
    oh1 = m2.astype(jnp.float32)
    ohsum = oh0 + oh1
    s_cum = jnp.dot(ltri_ref[...], ohsum, preferred_element_type=jnp.float32)

    carry = carry_ref[...]
    rank0 = jnp.sum((s_cum + carry) * oh0, axis=1, keepdims=True).astype(jnp.int32)
    rank1 = jnp.sum((s_cum + carry) * oh1, axis=1, keepdims=True).astype(jnp.int32)
    carry_ref[...] = carry + jnp.sum(ohsum, axis=0, keepdims=True)
    psum_ref[...] += jnp.sum(probs, axis=0, keepdims=True)

    er0 = (i1 << 13) | rank0
    er1 = (i2 << 13) | rank1
    mi_ref[...] = jnp.where(col == 0, er0, 0) + jnp.where(col == 1, er1, 0)
    mf_ref[...] = jnp.where(col == 0, w0, 0.0) + jnp.where(col == 1, w1, 0.0)

    @pl.when(i == nsteps - 1)
    def _():
        total = ntok * nsteps
        cnt = carry_ref[...]
        cnt_ref[...] = cnt
        fi = cnt / total
        pi = psum_ref[...] / total
        aux = 0.01 * NUM_EXPERTS * jnp.sum(fi * pi)
        aux_ref[...] = jnp.full((1, LANES), aux, dtype=jnp.float32)


def _router(x_flat, wr_pad, ltri):
    n = x_flat.shape[0]
    nsteps = n // ROUTER_CHUNK
    return pl.pallas_call(
        _router_body,
        grid=(nsteps,),
        in_specs=[
            pl.BlockSpec((ROUTER_CHUNK, D_MODEL), lambda i: (i, 0)),
            pl.BlockSpec((D_MODEL, LANES), lambda i: (0, 0)),
            pl.BlockSpec((ROUTER_CHUNK, ROUTER_CHUNK), lambda i: (0, 0)),
        ],
        out_specs=[
            pl.BlockSpec((ROUTER_CHUNK, LANES), lambda i: (i, 0)),
            pl.BlockSpec((ROUTER_CHUNK, LANES), lambda i: (i, 0)),
            pl.BlockSpec((1, LANES), lambda i: (0, 0)),
            pl.BlockSpec((1, LANES), lambda i: (0, 0)),
            pl.BlockSpec((ROUTER_CHUNK, D2), lambda i: (i, 0)),
        ],
        out_shape=[
            jax.ShapeDtypeStruct((n, LANES), jnp.int32),
            jax.ShapeDtypeStruct((n, LANES), jnp.float32),
            jax.ShapeDtypeStruct((1, LANES), jnp.float32),
            jax.ShapeDtypeStruct((1, LANES), jnp.float32),
            jax.ShapeDtypeStruct((n, D2), jnp.int32),
        ],
        scratch_shapes=[
            pltpu.VMEM((1, LANES), jnp.float32),
            pltpu.VMEM((1, LANES), jnp.float32),
        ],
    )(x_flat, wr_pad, ltri)


def _dispatch(x_flat, er0, er1, off16):
    n = x_flat.shape[0]
    tpw = n // NW
    sub = 64

    @functools.partial(
        pl.kernel,
        out_type=[
            jax.ShapeDtypeStruct((2 * n, D2), jnp.int32),
            jax.ShapeDtypeStruct((n,), jnp.int32),
            jax.ShapeDtypeStruct((n,), jnp.int32),
        ],
        mesh=plsc.VectorSubcoreMesh(core_axis_name="c", subcore_axis_name="s"),
        scratch_types=[
            pltpu.VMEM((16,), jnp.int32),
            pltpu.VMEM((sub,), jnp.int32),
            pltpu.VMEM((sub,), jnp.int32),
            pltpu.VMEM((sub,), jnp.int32),
            pltpu.VMEM((sub,), jnp.int32),
            pltpu.VMEM((sub, D2), jnp.int32),
            pltpu.SemaphoreType.DMA,
            pltpu.SemaphoreType.DMA,
            pltpu.SemaphoreType.DMA,
        ],
    )
    def k(x_hbm, er0_hbm, er1_hbm, offs_hbm,
          xs_hbm, p0_hbm, p1_hbm,
          off_v, er0_v, er1_v, idx0_v, idx1_v, xv, sem0, sem1, semx):
        wid = lax.axis_index("s") * NC + lax.axis_index("c")
        base = wid * tpw
        pltpu.sync_copy(offs_hbm, off_v)
        for j in range(tpw // sub):
            tb = base + j * sub
            cpx = pltpu.async_copy(x_hbm.at[pl.ds(tb, sub)], xv, semx)
            cp0 = pltpu.async_copy(er0_hbm.at[pl.ds(tb, sub)], er0_v, sem0)
            cp1 = pltpu.async_copy(er1_hbm.at[pl.ds(tb, sub)], er1_v, sem1)
            cp0.wait()
            cp1.wait()
            off_reg = off_v[...]
            for s in range(sub // L):
                sl = pl.ds(s * L, L)
                erv0 = er0_v[sl]
                erv1 = er1_v[sl]
                ev0 = erv0 >> 13
                ev1 = erv1 >> 13
                acc0 = erv0 & 8191
                acc1 = erv1 & 8191
                for kk in range(NUM_EXPERTS):
                    off_k = off_reg[kk]
                    acc0 = acc0 + jnp.where(ev0 == kk, off_k, 0)
                    acc1 = acc1 + jnp.where(ev1 == kk, off_k, 0)
                idx0_v[sl] = acc0
                idx1_v[sl] = acc1
            cpx.wait()
            cs0 = pltpu.async_copy(xv, xs_hbm.at[idx0_v], sem0)
            cs1 = pltpu.async_copy(xv, xs_hbm.at[idx1_v], sem1)
            cs0.wait()
            cs1.wait()
            pltpu.sync_copy(idx0_v, p0_hbm.at[pl.ds(tb, sub)])
            pltpu.sync_copy(idx1_v, p1_hbm.at[pl.ds(tb, sub)])

    return k(x_flat, er0, er1, off16)


def _gffn_body(se_ref, sm_ref, sa_ref, off_ref, xs_ref, w1_ref, w2_ref, out_ref,
               acc_ref):
    g = pl.program_id(0)
    nsteps = pl.num_programs(0)
    e = se_ref[g]
    m = sm_ref[g]
    prev_m = sm_ref[jnp.maximum(g - 1, 0)]
    first = (g == 0) | (m != prev_m)
    gn = jnp.minimum(g + 1, nsteps - 1)
    last = (g == nsteps - 1) | (sm_ref[gn] != m) | (sa_ref[gn] == 0)

    @pl.when(sa_ref[g] == 1)
    def _():
        packed = xs_ref[...]
        a = _unpack_lo(packed)
        bzz = _unpack_hi(packed)
        h = (jnp.dot(a, w1_ref[0, :D2, :], preferred_element_type=jnp.float32)
             + jnp.dot(bzz, w1_ref[0, D2:, :], preferred_element_type=jnp.float32))
        h = h * jax.nn.sigmoid(h)
        o = jnp.dot(h, w2_ref[0], preferred_element_type=jnp.float32)
        rowg = m * M + jax.lax.broadcasted_iota(jnp.int32, (M, 1), 0)
        mask = (rowg >= off_ref[e]) & (rowg < off_ref[e + 1])
        contrib = jnp.where(mask, o, 0.0)

        @pl.when(first)
        def _():
            acc_ref[...] = contrib

        @pl.when(jnp.logical_not(first))
        def _():
            acc_ref[...] += contrib

        @pl.when(last)
        def _():
            acc = acc_ref[...]
            out_ref[...] = _pack_halves(acc[:, :D2], acc[:, D2:])


def _gffn(xs32, w1b, w2b, step_e, step_m, step_act, off9, nt):
    rows = xs32.shape[0]
    grid_spec = pltpu.PrefetchScalarGridSpec(
        num_scalar_prefetch=4,
        grid=(nt,),
        in_specs=[
            pl.BlockSpec((M, D2), lambda g, se, sm, sa, off: (sm[g], 0)),
            pl.BlockSpec((1, D_MODEL, HIDDEN), lambda g, se, sm, sa, off: (se[g], 0, 0)),
            pl.BlockSpec((1, HIDDEN, D_MODEL), lambda g, se, sm, sa, off: (se[g], 0, 0)),
        ],
        out_specs=pl.BlockSpec((M, D2), lambda g, se, sm, sa, off: (sm[g], 0)),
        scratch_shapes=[pltpu.VMEM((M, D_MODEL), jnp.float32)],
    )
    return pl.pallas_call(
        _gffn_body,
        grid_spec=grid_spec,
        out_shape=jax.ShapeDtypeStruct((rows, D2), jnp.int32),
    )(step_e, step_m, step_act, off9, xs32, w1b, w2b)


def _permute(ys, p0, p1):
    n = p0.shape[0]
    tpw = n // NW
    sub = 64

    @functools.partial(
        pl.kernel,
        out_type=[
            jax.ShapeDtypeStruct((n, D2), jnp.int32),
            jax.ShapeDtypeStruct((n, D2), jnp.int32),
        ],
        mesh=plsc.VectorSubcoreMesh(core_axis_name="c", subcore_axis_name="s"),
        scratch_types=[
            pltpu.VMEM((sub,), jnp.int32),
            pltpu.VMEM((sub,), jnp.int32),
            pltpu.VMEM((sub, D2), jnp.int32),
            pltpu.VMEM((sub, D2), jnp.int32),
            pltpu.SemaphoreType.DMA,
            pltpu.SemaphoreType.DMA,
        ],
    )
    def k(ys_hbm, p0_hbm, p1_hbm, ya_hbm, yb_hbm,
          p0_v, p1_v, ya, yb, sem0, sem1):
        wid = lax.axis_index("s") * NC + lax.axis_index("c")
        base = wid * tpw
        for j in range(tpw // sub):
            tb = base + j * sub
            pltpu.sync_copy(p0_hbm.at[pl.ds(tb, sub)], p0_v)
            pltpu.sync_copy(p1_hbm.at[pl.ds(tb, sub)], p1_v)
            cpa = pltpu.async_copy(ys_hbm.at[p0_v], ya, sem0)
            cpb = pltpu.async_copy(ys_hbm.at[p1_v], yb, sem1)
            cpa.wait()
            cpb.wait()
            pltpu.sync_copy(ya, ya_hbm.at[pl.ds(tb, sub)])
            pltpu.sync_copy(yb, yb_hbm.at[pl.ds(tb, sub)])

    return k(ys, p0, p1)


def _wsum_body(ya_ref, yb_ref, mf_ref, out_ref):
    w0 = mf_ref[:, 0:1]
    w1 = mf_ref[:, 1:2]
    ya = ya_ref[...]
    yb = yb_ref[...]
    out_ref[:, :D2] = _unpack_lo(ya) * w0 + _unpack_lo(yb) * w1
    out_ref[:, D2:] = _unpack_hi(ya) * w0 + _unpack_hi(yb) * w1


def _wsum(ya32, yb32, mf):
    n = ya32.shape[0]
    chunk = 512
    return pl.pallas_call(
        _wsum_body,
        grid=(n // chunk,),
        in_specs=[
            pl.BlockSpec((chunk, D2), lambda i: (i, 0)),
            pl.BlockSpec((chunk, D2), lambda i: (i, 0)),
            pl.BlockSpec((chunk, LANES), lambda i: (i, 0)),
        ],
        out_specs=pl.BlockSpec((chunk, D_MODEL), lambda i: (i, 0)),
        out_shape=jax.ShapeDtypeStruct((n, D_MODEL), jnp.float32),
    )(ya32, yb32, mf)


def kernel(x, Wr, W1, W2):
    b, t, c = x.shape
    n = b * t
    x_flat = x.reshape(n, c)
    wr_pad = jnp.pad(Wr, ((0, 0), (0, LANES - NUM_EXPERTS)))

    ltri = jnp.tril(jnp.ones((ROUTER_CHUNK, ROUTER_CHUNK), jnp.float32), -1)
    mi, mf, cnt_row, aux_vec, xb = _router(x_flat, wr_pad, ltri)
    er0 = mi[:, 0]
    er1 = mi[:, 1]
    counts = cnt_row[0, :NUM_EXPERTS].astype(jnp.int32)

    off9 = jnp.concatenate([jnp.zeros((1,), jnp.int32), jnp.cumsum(counts)])
    off16 = jnp.pad(off9, (0, 16 - off9.shape[0]))

    mt = (2 * n) // M
    nt = mt + NUM_EXPERTS - 1
    lo_row = off9[:NUM_EXPERTS]
    hi_row = off9[1:]
    tile_lo = lo_row // M
    tile_last = jnp.where(counts > 0, (hi_row - 1) // M, tile_lo)
    nact = jnp.where(counts > 0, tile_last - tile_lo + 1, 0)
    gstart = jnp.concatenate([jnp.zeros((1,), jnp.int32),
                              jnp.cumsum(nact)[:-1]])
    g = jnp.arange(nt, dtype=jnp.int32)
    e_of_g = jnp.sum((g[:, None] >= gstart[None, :]).astype(jnp.int32), axis=1) - 1
    within = g - gstart[e_of_g]
    act = within < nact[e_of_g]
    m_of_g = jnp.where(act, tile_lo[e_of_g] + within, mt - 1)
    step_e = e_of_g.astype(jnp.int32)
    step_m = m_of_g.astype(jnp.int32)
    step_act = act.astype(jnp.int32)

    xs32, p0, p1 = _dispatch(xb, er0, er1, off16)
    ys32 = _gffn(xs32, W1, W2, step_e, step_m, step_act, off9, nt)
    ya32, yb32 = _permute(ys32, p0, p1)
    out_flat = _wsum(ya32, yb32, mf)
    return out_flat.reshape(b, t, c), aux_vec[0, 0]
